# baseline (speedup 1.0000x reference)
import jax
import jax.numpy as jnp
from jax import lax
from jax.experimental import pallas as pl
from jax.experimental.pallas import tpu as pltpu
from jax.experimental.pallas import tpu_sc as plsc

D = 32
L = 200
B = 4096
N = B * L
V = 1000000
NC, NS = 2, 16
NW = NC * NS
BPW = B // NW
PER_W = N // NW
RPC = 8
CHUNK = RPC * L
NCH = BPW // RPC
SUB = 80
NSUB = CHUNK // SUB
LANES = 16
_COLS = [k * LANES for k in range(L // LANES)] + [L - LANES]


def _flatten_body(x_hbm, out_hbm, xb, xf, sem):
    wid = lax.axis_index("s") * NC + lax.axis_index("c")
    b0 = pl.multiple_of(wid * BPW, BPW)
    pltpu.sync_copy(x_hbm.at[pl.ds(b0, BPW)], xb)

    def row(r, _):
        for col in _COLS:
            xf[pl.ds(r * L + col, LANES)] = xb[r, pl.ds(col, LANES)]
        return 0

    lax.fori_loop(0, BPW, row, 0)
    pltpu.sync_copy(xf, out_hbm.at[pl.ds(wid * PER_W, PER_W)])


VPW = V // NW
TCH = 1568
TNC = 20
TCF = TCH + 8


def _transpose_body(tt_hbm, out_hbm, colbuf, rowbuf, sem):
    wid = lax.axis_index("s") * NC + lax.axis_index("c")
    start = wid * VPW

    def chunk(ck, _):
        r0 = jnp.minimum(start + ck * TCH, V - TCH)
        r0a = pl.multiple_of((r0 // 8) * 8, 8)
        sh = r0 - r0a
        for d in range(D):
            pltpu.sync_copy(tt_hbm.at[d, pl.ds(r0a, TCF)], colbuf.at[d])
        rowblock = lax.broadcasted_iota(jnp.int32, (LANES,), 0)
        for d in range(D):
            cols = jnp.full((LANES,), d, jnp.int32)

            def kstep(k, _):
                vals = colbuf[d, pl.ds(k * LANES + sh, LANES)]
                plsc.store_scatter(rowbuf, [rowblock + k * LANES, cols], vals)
                return 0

            lax.fori_loop(0, TCH // LANES, kstep, 0)
        pltpu.sync_copy(rowbuf, out_hbm.at[pl.ds(r0, TCH)])
        return 0

    lax.fori_loop(0, TNC, chunk, 0)


def _gather_body(x_hbm, table_hbm, pos_hbm, out_hbm, idx_v, gbuf, dest_v,
                 pos_v, sem):
    wid = lax.axis_index("s") * NC + lax.axis_index("c")
    base_w = pl.multiple_of(wid * PER_W, PER_W)
    pltpu.sync_copy(pos_hbm, pos_v)

    def chunk_body(ch, _):
        base = pl.multiple_of(base_w + ch * CHUNK, CHUNK)
        pltpu.sync_copy(x_hbm.at[pl.ds(base, CHUNK)], idx_v)
        copies = [
            pltpu.async_copy(
                table_hbm.at[idx_v.at[pl.ds(j * SUB, SUB)]],
                gbuf.at[pl.ds(j * SUB, SUB)],
                sem,
            )
            for j in range(NSUB)
        ]
        for cp in copies:
            cp.wait()

        def add_l(l, _):
            p0 = pos_v[l, pl.ds(0, LANES)]
            p1 = pos_v[l, pl.ds(LANES, LANES)]
            for t in range(RPC):
                r2 = t * L + l
                dest_v[pl.ds(r2 * D, LANES)] = gbuf[r2, pl.ds(0, LANES)] + p0
                dest_v[pl.ds(r2 * D + LANES, LANES)] = (
                    gbuf[r2, pl.ds(LANES, LANES)] + p1)
            return 0

        lax.fori_loop(0, L, add_l, 0)
        pltpu.sync_copy(dest_v, out_hbm.at[pl.ds(base * D, CHUNK * D)])
        return 0

    lax.fori_loop(0, NCH, chunk_body, 0)


_mesh = plsc.VectorSubcoreMesh(core_axis_name="c", subcore_axis_name="s")

_flatten = pl.kernel(
    _flatten_body,
    out_type=jax.ShapeDtypeStruct((N,), jnp.int32),
    mesh=_mesh,
    scratch_types=[
        pltpu.VMEM((BPW, L), jnp.int32),
        pltpu.VMEM((PER_W,), jnp.int32),
        pltpu.SemaphoreType.DMA,
    ],
)

_transpose = pl.kernel(
    _transpose_body,
    out_type=jax.ShapeDtypeStruct((V, D), jnp.float32),
    mesh=_mesh,
    scratch_types=[
        pltpu.VMEM((D, TCF), jnp.float32),
        pltpu.VMEM((TCH, D), jnp.float32),
        pltpu.SemaphoreType.DMA,
    ],
    compiler_params=pltpu.CompilerParams(
        use_tc_tiling_on_sc=False, needs_layout_passes=False),
)

_gather = pl.kernel(
    _gather_body,
    out_type=jax.ShapeDtypeStruct((N * D,), jnp.float32),
    mesh=_mesh,
    scratch_types=[
        pltpu.VMEM((CHUNK,), jnp.int32),
        pltpu.VMEM((CHUNK, D), jnp.float32),
        pltpu.VMEM((CHUNK * D,), jnp.float32),
        pltpu.VMEM((L, D), jnp.float32),
        pltpu.SemaphoreType.DMA,
    ],
    compiler_params=pltpu.CompilerParams(use_tc_tiling_on_sc=False),
)


@jax.jit
def _run(x, table, pos):
    xf = _flatten(x)
    tl = _transpose(jnp.transpose(table))
    out = _gather(xf, tl, pos)
    return out.reshape(B, L, D)


def kernel(x, table, pos):
    return _run(x, table, pos)


# speedup vs baseline: 3.1187x; 3.1187x over previous
import jax
import jax.numpy as jnp
from jax import lax
from jax.experimental import pallas as pl
from jax.experimental.pallas import tpu as pltpu
from jax.experimental.pallas import tpu_sc as plsc

D = 32
L = 200
B = 4096
N = B * L
V = 1000000
NC, NS = 2, 16
NW = NC * NS
BPW = B // NW
PER_W = N // NW
RPC = 8
CHUNK = RPC * L
NCH = BPW // RPC
SUB = 80
NSUB = CHUNK // SUB
LANES = 16
_COLS = [k * LANES for k in range(L // LANES)] + [L - LANES]


def _flatten_body(x_hbm, out_hbm, xb, xf, sem):
    wid = lax.axis_index("s") * NC + lax.axis_index("c")
    b0 = pl.multiple_of(wid * BPW, BPW)
    pltpu.sync_copy(x_hbm.at[pl.ds(b0, BPW)], xb)

    def row(r, _):
        for col in _COLS:
            xf[pl.ds(r * L + col, LANES)] = xb[r, pl.ds(col, LANES)]
        return 0

    lax.fori_loop(0, BPW, row, 0)
    pltpu.sync_copy(xf, out_hbm.at[pl.ds(wid * PER_W, PER_W)])


VPW = V // NW
DCH = 704
NCHD = 45


def _depad_body(t_hbm, out_hbm, tbuf, ubuf, sem):
    wid = lax.axis_index("s") * NC + lax.axis_index("c")
    start8 = pl.multiple_of((wid * VPW) // 8 * 8, 8)

    def chunk(c, _):
        r0 = pl.multiple_of(jnp.minimum(start8 + c * DCH, V - DCH), 8)
        pltpu.sync_copy(t_hbm.at[pl.ds(r0, DCH)], tbuf)

        def rows(r, _):
            for h in range(2):
                ubuf[pl.ds(r * D + h * LANES, LANES)] = (
                    tbuf[r, pl.ds(h * LANES, LANES)])
            return 0

        lax.fori_loop(0, DCH, rows, 0)
        pltpu.sync_copy(ubuf, out_hbm.at[pl.ds(r0 * D, DCH * D)])
        return 0

    lax.fori_loop(0, NCHD, chunk, 0)


def _gather_body(x_hbm, table_hbm, pos_hbm, out_hbm, idx_v, gbuf, dest_v,
                 pos_v, sem):
    wid = lax.axis_index("s") * NC + lax.axis_index("c")
    base_w = pl.multiple_of(wid * PER_W, PER_W)
    pltpu.sync_copy(pos_hbm, pos_v)

    def chunk_body(ch, _):
        base = pl.multiple_of(base_w + ch * CHUNK, CHUNK)
        pltpu.sync_copy(x_hbm.at[pl.ds(base, CHUNK)], idx_v)
        copies = [
            pltpu.async_copy(
                table_hbm.at[idx_v.at[pl.ds(j * SUB, SUB)]],
                gbuf.at[pl.ds(j * SUB, SUB)],
                sem,
            )
            for j in range(NSUB)
        ]
        for cp in copies:
            cp.wait()

        def add_l(l, _):
            p0 = pos_v[l, pl.ds(0, LANES)]
            p1 = pos_v[l, pl.ds(LANES, LANES)]
            for t in range(RPC):
                r2 = t * L + l
                dest_v[pl.ds(r2 * D, LANES)] = gbuf[r2, pl.ds(0, LANES)] + p0
                dest_v[pl.ds(r2 * D + LANES, LANES)] = (
                    gbuf[r2, pl.ds(LANES, LANES)] + p1)
            return 0

        lax.fori_loop(0, L, add_l, 0)
        pltpu.sync_copy(dest_v, out_hbm.at[pl.ds(base * D, CHUNK * D)])
        return 0

    lax.fori_loop(0, NCH, chunk_body, 0)


_mesh = plsc.VectorSubcoreMesh(core_axis_name="c", subcore_axis_name="s")

_flatten = pl.kernel(
    _flatten_body,
    out_type=jax.ShapeDtypeStruct((N,), jnp.int32),
    mesh=_mesh,
    scratch_types=[
        pltpu.VMEM((BPW, L), jnp.int32),
        pltpu.VMEM((PER_W,), jnp.int32),
        pltpu.SemaphoreType.DMA,
    ],
)

_depad = pl.kernel(
    _depad_body,
    out_type=jax.ShapeDtypeStruct((V * D,), jnp.float32),
    mesh=_mesh,
    scratch_types=[
        pltpu.VMEM((DCH, D), jnp.float32),
        pltpu.VMEM((DCH * D,), jnp.float32),
        pltpu.SemaphoreType.DMA,
    ],
)

_gather = pl.kernel(
    _gather_body,
    out_type=jax.ShapeDtypeStruct((N * D,), jnp.float32),
    mesh=_mesh,
    scratch_types=[
        pltpu.VMEM((CHUNK,), jnp.int32),
        pltpu.VMEM((CHUNK, D), jnp.float32),
        pltpu.VMEM((CHUNK * D,), jnp.float32),
        pltpu.VMEM((L, D), jnp.float32),
        pltpu.SemaphoreType.DMA,
    ],
    compiler_params=pltpu.CompilerParams(use_tc_tiling_on_sc=False),
)


@jax.jit
def _run(x, table, pos):
    xf = _flatten(x)
    tl = _depad(table).reshape(V, D)
    out = _gather(xf, tl, pos)
    return out.reshape(B, L, D)


def kernel(x, table, pos):
    return _run(x, table, pos)


# speedup vs baseline: 3.1694x; 1.0163x over previous
import jax
import jax.numpy as jnp
from jax import lax
from jax.experimental import pallas as pl
from jax.experimental.pallas import tpu as pltpu
from jax.experimental.pallas import tpu_sc as plsc

D = 32
L = 200
B = 4096
N = B * L
V = 1000000
NC, NS = 2, 16
NW = NC * NS
BPW = B // NW
PER_W = N // NW
RPC = 8
CHUNK = RPC * L
NCH = BPW // RPC
SUB = 80
NSUB = CHUNK // SUB
LANES = 16
_COLS = [k * LANES for k in range(L // LANES)] + [L - LANES]


def _flatten_body(x_hbm, out_hbm, xb, xf, sem):
    wid = lax.axis_index("s") * NC + lax.axis_index("c")
    b0 = pl.multiple_of(wid * BPW, BPW)
    pltpu.sync_copy(x_hbm.at[pl.ds(b0, BPW)], xb)

    def row(r, _):
        for col in _COLS:
            xf[pl.ds(r * L + col, LANES)] = xb[r, pl.ds(col, LANES)]
        return 0

    lax.fori_loop(0, BPW, row, 0)
    pltpu.sync_copy(xf, out_hbm.at[pl.ds(wid * PER_W, PER_W)])


VPW = V // NW
CH2 = 1536
CF2 = CH2 + 128
NCH2 = 21


def _transpose_body(tt_hbm, out_hbm, tbuf, ubuf, sem):
    wid = lax.axis_index("s") * NC + lax.axis_index("c")
    start = wid * VPW
    rowidx = lax.broadcasted_iota(jnp.int32, (LANES,), 0) * D

    def chunk(c, _):
        v0 = jnp.minimum(start + c * CH2, V - CH2)
        v0a = pl.multiple_of(jnp.minimum((v0 // 128) * 128, V - CF2), 128)
        sh = v0 - v0a
        pltpu.sync_copy(tt_hbm.at[:, pl.ds(v0a, CF2)], tbuf)
        for d in range(D):

            def kstep(k, _):
                vals = tbuf[d, pl.ds(k * LANES + sh, LANES)]
                plsc.store_scatter(ubuf, [rowidx + (k * (LANES * D) + d)], vals)
                return 0

            lax.fori_loop(0, CH2 // LANES, kstep, 0)
        pltpu.sync_copy(ubuf, out_hbm.at[pl.ds(v0 * D, CH2 * D)])
        return 0

    lax.fori_loop(0, NCH2, chunk, 0)


def _gather_body(x_hbm, table_hbm, pos_hbm, out_hbm, idx_v, gbuf, dest_v,
                 pos_v, sem):
    wid = lax.axis_index("s") * NC + lax.axis_index("c")
    base_w = pl.multiple_of(wid * PER_W, PER_W)
    pltpu.sync_copy(pos_hbm, pos_v)

    def chunk_body(ch, _):
        base = pl.multiple_of(base_w + ch * CHUNK, CHUNK)
        pltpu.sync_copy(x_hbm.at[pl.ds(base, CHUNK)], idx_v)
        copies = [
            pltpu.async_copy(
                table_hbm.at[idx_v.at[pl.ds(j * SUB, SUB)]],
                gbuf.at[pl.ds(j * SUB, SUB)],
                sem,
            )
            for j in range(NSUB)
        ]
        for cp in copies:
            cp.wait()

        def add_l(l, _):
            p0 = pos_v[l, pl.ds(0, LANES)]
            p1 = pos_v[l, pl.ds(LANES, LANES)]
            for t in range(RPC):
                r2 = t * L + l
                dest_v[pl.ds(r2 * D, LANES)] = gbuf[r2, pl.ds(0, LANES)] + p0
                dest_v[pl.ds(r2 * D + LANES, LANES)] = (
                    gbuf[r2, pl.ds(LANES, LANES)] + p1)
            return 0

        lax.fori_loop(0, L, add_l, 0)
        pltpu.sync_copy(dest_v, out_hbm.at[pl.ds(base * D, CHUNK * D)])
        return 0

    lax.fori_loop(0, NCH, chunk_body, 0)


_mesh = plsc.VectorSubcoreMesh(core_axis_name="c", subcore_axis_name="s")

_flatten = pl.kernel(
    _flatten_body,
    out_type=jax.ShapeDtypeStruct((N,), jnp.int32),
    mesh=_mesh,
    scratch_types=[
        pltpu.VMEM((BPW, L), jnp.int32),
        pltpu.VMEM((PER_W,), jnp.int32),
        pltpu.SemaphoreType.DMA,
    ],
)

_transpose = pl.kernel(
    _transpose_body,
    out_type=jax.ShapeDtypeStruct((V * D,), jnp.float32),
    mesh=_mesh,
    scratch_types=[
        pltpu.VMEM((D, CF2), jnp.float32),
        pltpu.VMEM((CH2 * D,), jnp.float32),
        pltpu.SemaphoreType.DMA,
    ],
    compiler_params=pltpu.CompilerParams(needs_layout_passes=False),
)

_gather = pl.kernel(
    _gather_body,
    out_type=jax.ShapeDtypeStruct((N * D,), jnp.float32),
    mesh=_mesh,
    scratch_types=[
        pltpu.VMEM((CHUNK,), jnp.int32),
        pltpu.VMEM((CHUNK, D), jnp.float32),
        pltpu.VMEM((CHUNK * D,), jnp.float32),
        pltpu.VMEM((L, D), jnp.float32),
        pltpu.SemaphoreType.DMA,
    ],
    compiler_params=pltpu.CompilerParams(use_tc_tiling_on_sc=False),
)


@jax.jit
def _run(x, table, pos):
    xf = _flatten(x)
    tl = _transpose(jnp.transpose(table)).reshape(V, D)
    out = _gather(xf, tl, pos)
    return out.reshape(B, L, D)


def kernel(x, table, pos):
    return _run(x, table, pos)


# speedup vs baseline: 3.1783x; 1.0028x over previous
import jax
import jax.numpy as jnp
from jax import lax
from jax.experimental import pallas as pl
from jax.experimental.pallas import tpu as pltpu
from jax.experimental.pallas import tpu_sc as plsc

D = 32
L = 200
B = 4096
N = B * L
V = 1000000
NC, NS = 2, 16
NW = NC * NS
BPW = B // NW
PER_W = N // NW
RPC = 8
CHUNK = RPC * L
NCH = BPW // RPC
SUB = 80
NSUB = CHUNK // SUB
LANES = 16
_COLS = [k * LANES for k in range(L // LANES)] + [L - LANES]


def _flatten_body(x_hbm, out_hbm, xb, xf, sem):
    wid = lax.axis_index("s") * NC + lax.axis_index("c")
    b0 = pl.multiple_of(wid * BPW, BPW)
    pltpu.sync_copy(x_hbm.at[pl.ds(b0, BPW)], xb)

    def row(r, _):
        for col in _COLS:
            xf[pl.ds(r * L + col, LANES)] = xb[r, pl.ds(col, LANES)]
        return 0

    lax.fori_loop(0, BPW, row, 0)
    pltpu.sync_copy(xf, out_hbm.at[pl.ds(wid * PER_W, PER_W)])


VPW = V // NW
CH2 = 1536
NCH2 = 21
VA = (V // 128) * 128
VT = V - VA


def _transpose_body(tt_hbm, tail_hbm, out_hbm, tbuf, tailbuf, ubuf, sem):
    wid = lax.axis_index("s") * NC + lax.axis_index("c")
    start = pl.multiple_of((wid * VPW) // 128 * 128, 128)
    rowidx = lax.broadcasted_iota(jnp.int32, (LANES,), 0) * D

    def chunk(c, _):
        v0 = pl.multiple_of(jnp.minimum(start + c * CH2, VA - CH2), 128)
        pltpu.sync_copy(tt_hbm.at[:, pl.ds(v0, CH2)], tbuf)
        for d in range(D):

            def kstep(k, _):
                vals = tbuf[d, pl.ds(k * LANES, LANES)]
                plsc.store_scatter(ubuf, [rowidx + (k * (LANES * D) + d)], vals)
                return 0

            lax.fori_loop(0, CH2 // LANES, kstep, 0)
        pltpu.sync_copy(ubuf, out_hbm.at[pl.ds(v0 * D, CH2 * D)])
        return 0

    lax.fori_loop(0, NCH2, chunk, 0)

    @pl.when(wid == NW - 1)
    def _tail():
        pltpu.sync_copy(tail_hbm, tailbuf)
        for r in range(VT):
            for h in range(2):
                ubuf[pl.ds(r * D + h * LANES, LANES)] = (
                    tailbuf[r, pl.ds(h * LANES, LANES)])
        pltpu.sync_copy(
            ubuf.at[pl.ds(0, VT * D)],
            out_hbm.at[pl.ds(VA * D, VT * D)],
        )


def _gather_body(x_hbm, table_hbm, pos_hbm, out_hbm, idx_v, gbuf, dest_v,
                 pos_v, sem):
    wid = lax.axis_index("s") * NC + lax.axis_index("c")
    base_w = pl.multiple_of(wid * PER_W, PER_W)
    pltpu.sync_copy(pos_hbm, pos_v)

    def chunk_body(ch, _):
        base = pl.multiple_of(base_w + ch * CHUNK, CHUNK)
        pltpu.sync_copy(x_hbm.at[pl.ds(base, CHUNK)], idx_v)
        copies = [
            pltpu.async_copy(
                table_hbm.at[idx_v.at[pl.ds(j * SUB, SUB)]],
                gbuf.at[pl.ds(j * SUB, SUB)],
                sem,
            )
            for j in range(NSUB)
        ]
        for cp in copies:
            cp.wait()

        def add_l(l, _):
            p0 = pos_v[l, pl.ds(0, LANES)]
            p1 = pos_v[l, pl.ds(LANES, LANES)]
            for t in range(RPC):
                r2 = t * L + l
                dest_v[pl.ds(r2 * D, LANES)] = gbuf[r2, pl.ds(0, LANES)] + p0
                dest_v[pl.ds(r2 * D + LANES, LANES)] = (
                    gbuf[r2, pl.ds(LANES, LANES)] + p1)
            return 0

        lax.fori_loop(0, L, add_l, 0)
        pltpu.sync_copy(dest_v, out_hbm.at[pl.ds(base * D, CHUNK * D)])
        return 0

    lax.fori_loop(0, NCH, chunk_body, 0)


_mesh = plsc.VectorSubcoreMesh(core_axis_name="c", subcore_axis_name="s")

_flatten = pl.kernel(
    _flatten_body,
    out_type=jax.ShapeDtypeStruct((N,), jnp.int32),
    mesh=_mesh,
    scratch_types=[
        pltpu.VMEM((BPW, L), jnp.int32),
        pltpu.VMEM((PER_W,), jnp.int32),
        pltpu.SemaphoreType.DMA,
    ],
)

_transpose = pl.kernel(
    _transpose_body,
    out_type=jax.ShapeDtypeStruct((V * D,), jnp.float32),
    mesh=_mesh,
    scratch_types=[
        pltpu.VMEM((D, CH2), jnp.float32),
        pltpu.VMEM((VT, D), jnp.float32),
        pltpu.VMEM((CH2 * D,), jnp.float32),
        pltpu.SemaphoreType.DMA,
    ],
    compiler_params=pltpu.CompilerParams(needs_layout_passes=False),
)

_gather = pl.kernel(
    _gather_body,
    out_type=jax.ShapeDtypeStruct((N * D,), jnp.float32),
    mesh=_mesh,
    scratch_types=[
        pltpu.VMEM((CHUNK,), jnp.int32),
        pltpu.VMEM((CHUNK, D), jnp.float32),
        pltpu.VMEM((CHUNK * D,), jnp.float32),
        pltpu.VMEM((L, D), jnp.float32),
        pltpu.SemaphoreType.DMA,
    ],
    compiler_params=pltpu.CompilerParams(use_tc_tiling_on_sc=False),
)


@jax.jit
def _run(x, table, pos):
    xf = _flatten(x)
    tail = lax.slice(table, (VA, 0), (V, D))
    tl = _transpose(jnp.transpose(table), tail).reshape(V, D)
    out = _gather(xf, tl, pos)
    return out.reshape(B, L, D)


def kernel(x, table, pos):
    return _run(x, table, pos)


# speedup vs baseline: 3.9591x; 1.2457x over previous
import jax
import jax.numpy as jnp
from jax import lax
from jax.experimental import pallas as pl
from jax.experimental.pallas import tpu as pltpu
from jax.experimental.pallas import tpu_sc as plsc

D = 32
L = 200
B = 4096
N = B * L
V = 1000000
NC, NS = 2, 16
NW = NC * NS
BPW = B // NW
PER_W = N // NW
RPC = 8
CHUNK = RPC * L
NCH = BPW // RPC
SUB = 80
NSUB = CHUNK // SUB
LANES = 16
_COLS = [k * LANES for k in range(L // LANES)] + [L - LANES]


def _flatten_body(x_hbm, out_hbm, xb, xf, sem):
    wid = lax.axis_index("s") * NC + lax.axis_index("c")
    b0 = pl.multiple_of(wid * BPW, BPW)
    pltpu.sync_copy(x_hbm.at[pl.ds(b0, BPW)], xb)

    def row(r, _):
        for col in _COLS:
            xf[pl.ds(r * L + col, LANES)] = xb[r, pl.ds(col, LANES)]
        return 0

    lax.fori_loop(0, BPW, row, 0)
    pltpu.sync_copy(xf, out_hbm.at[pl.ds(wid * PER_W, PER_W)])


def _gather_body(x_hbm, table_hbm, pos_hbm, out_hbm, idx_v, gbuf, dest_v,
                 pos_v, sem):
    wid = lax.axis_index("s") * NC + lax.axis_index("c")
    base_w = pl.multiple_of(wid * PER_W, PER_W)
    pltpu.sync_copy(pos_hbm, pos_v)

    def chunk_body(ch, _):
        base = pl.multiple_of(base_w + ch * CHUNK, CHUNK)
        pltpu.sync_copy(x_hbm.at[pl.ds(base, CHUNK)], idx_v)
        copies = [
            pltpu.async_copy(
                table_hbm.at[idx_v.at[pl.ds(j * SUB, SUB)]],
                gbuf.at[pl.ds(j * SUB, SUB)],
                sem,
            )
            for j in range(NSUB)
        ]
        for cp in copies:
            cp.wait()

        def add_l(l, _):
            p0 = pos_v[l, pl.ds(0, LANES)]
            p1 = pos_v[l, pl.ds(LANES, LANES)]
            for t in range(RPC):
                r2 = t * L + l
                dest_v[pl.ds(r2 * D, LANES)] = gbuf[r2, pl.ds(0, LANES)] + p0
                dest_v[pl.ds(r2 * D + LANES, LANES)] = (
                    gbuf[r2, pl.ds(LANES, LANES)] + p1)
            return 0

        lax.fori_loop(0, L, add_l, 0)
        pltpu.sync_copy(dest_v, out_hbm.at[pl.ds(base * D, CHUNK * D)])
        return 0

    lax.fori_loop(0, NCH, chunk_body, 0)


_mesh = plsc.VectorSubcoreMesh(core_axis_name="c", subcore_axis_name="s")

_flatten = pl.kernel(
    _flatten_body,
    out_type=jax.ShapeDtypeStruct((N,), jnp.int32),
    mesh=_mesh,
    scratch_types=[
        pltpu.VMEM((BPW, L), jnp.int32),
        pltpu.VMEM((PER_W,), jnp.int32),
        pltpu.SemaphoreType.DMA,
    ],
)

_gather = pl.kernel(
    _gather_body,
    out_type=jax.ShapeDtypeStruct((N * D,), jnp.float32),
    mesh=_mesh,
    scratch_types=[
        pltpu.VMEM((CHUNK,), jnp.int32),
        pltpu.VMEM((CHUNK, D), jnp.float32),
        pltpu.VMEM((CHUNK * D,), jnp.float32),
        pltpu.VMEM((L, D), jnp.float32),
        pltpu.SemaphoreType.DMA,
    ],
    compiler_params=pltpu.CompilerParams(use_tc_tiling_on_sc=False),
)


@jax.jit
def _run(x, table, pos):
    xf = _flatten(x)
    out = _gather(xf, table, pos)
    return out.reshape(B, L, D)


def kernel(x, table, pos):
    return _run(x, table, pos)
